# baseline (speedup 1.0000x reference)
import functools

import jax
import jax.numpy as jnp
from jax import lax
from jax.experimental import pallas as pl
from jax.experimental.pallas import tpu as pltpu
from jax.experimental.pallas import tpu_sc as plsc

_NC, _NS, _L = 2, 16, 16
_NW = _NC * _NS
_ROWS, _D = 8192, 4096
_RPW = _ROWS // _NW
_R = 8
_HR = _R // 2
_NCHUNK = _RPW // _R
_NHALF = _RPW // _HR
_NF = _D // _L


def _sc_body(x_hbm, gidx_hbm, mask_hbm, out_hbm,
             idx_v, mask_v, ibuf_a, ibuf_b, gbuf_a, gbuf_b,
             sem_ia, sem_ib, sem_ga, sem_gb, sem_oa, sem_ob):
    ibufs, sem_i = (ibuf_a, ibuf_b), (sem_ia, sem_ib)
    gbufs, sem_g = (gbuf_a, gbuf_b), (sem_ga, sem_gb)
    sem_o = (sem_oa, sem_ob)

    wid = lax.axis_index("s") * _NC + lax.axis_index("c")
    wbase = wid * _RPW
    batch = wid // (_NW // 4)

    pltpu.sync_copy(gidx_hbm.at[pl.ds(wid * _NHALF, _NHALF)], idx_v)
    pltpu.sync_copy(mask_hbm.at[batch], mask_v)

    def patch(obuf, gb0, gb1):
        @plsc.parallel_loop(0, _NF, unroll=4)
        def feat(f):
            m = mask_v[pl.ds(f * _L, _L)]
            pred = m < 0.5
            col = lax.iota(jnp.int32, _L) + f * _L
            for h, gb in ((0, gb0), (1, gb1)):
                for r in range(_HR):
                    g = gb[r, pl.ds(f * _L, _L)]
                    row = jnp.full((_L,), _HR * h + r, dtype=jnp.int32)
                    plsc.store_scatter(obuf, [row, col], g, mask=pred)

    def issue_in(c, bs):
        base = wbase + c * _R
        pltpu.async_copy(x_hbm.at[pl.ds(base, _R)], ibufs[bs], sem_i[bs])
        pltpu.async_copy(x_hbm.at[idx_v.at[2 * c]], gbufs[0], sem_g[0])
        pltpu.async_copy(x_hbm.at[idx_v.at[2 * c + 1]], gbufs[1], sem_g[1])

    issue_in(0, 0)

    def pair(i, carry):
        for bs in (0, 1):
            c = 2 * i + bs
            base = wbase + c * _R
            nb = 1 - bs
            pltpu.make_async_copy(
                x_hbm.at[pl.ds(base, _R)], ibufs[bs], sem_i[bs]).wait()
            pltpu.make_async_copy(
                x_hbm.at[idx_v.at[2 * c]], gbufs[0], sem_g[0]).wait()
            pltpu.make_async_copy(
                x_hbm.at[idx_v.at[2 * c + 1]], gbufs[1], sem_g[1]).wait()
            patch(ibufs[bs], gbufs[0], gbufs[1])
            pltpu.async_copy(ibufs[bs], out_hbm.at[pl.ds(base, _R)],
                             sem_o[bs])

            @pl.when(c + 1 < _NCHUNK)
            def _issue_next():
                @pl.when(c >= 1)
                def _drain_prev_out():
                    pltpu.make_async_copy(
                        ibufs[nb],
                        out_hbm.at[pl.ds(wbase + (c - 1) * _R, _R)],
                        sem_o[nb]).wait()
                issue_in(c + 1, nb)
        return carry

    lax.fori_loop(0, _NCHUNK // 2, pair, 0)

    pltpu.make_async_copy(
        ibufs[0], out_hbm.at[pl.ds(wbase + (_NCHUNK - 2) * _R, _R)],
        sem_o[0]).wait()
    pltpu.make_async_copy(
        ibufs[1], out_hbm.at[pl.ds(wbase + (_NCHUNK - 1) * _R, _R)],
        sem_o[1]).wait()


@functools.cache
def _build():
    mesh = plsc.VectorSubcoreMesh(core_axis_name="c", subcore_axis_name="s")
    return pl.kernel(
        _sc_body,
        out_type=jax.ShapeDtypeStruct((_ROWS, _D), jnp.float32),
        mesh=mesh,
        scratch_types=[
            pltpu.VMEM((_NHALF, _HR), jnp.int32),
            pltpu.VMEM((_D,), jnp.float32),
            pltpu.VMEM((_R, _D), jnp.float32),
            pltpu.VMEM((_R, _D), jnp.float32),
            pltpu.VMEM((_HR, _D), jnp.float32),
            pltpu.VMEM((_HR, _D), jnp.float32),
            pltpu.SemaphoreType.DMA,
            pltpu.SemaphoreType.DMA,
            pltpu.SemaphoreType.DMA,
            pltpu.SemaphoreType.DMA,
            pltpu.SemaphoreType.DMA,
            pltpu.SemaphoreType.DMA,
        ],
        compiler_params=pltpu.CompilerParams(
            use_tc_tiling_on_sc=True, needs_layout_passes=False),
    )


@functools.cache
def _constants(bsz, seqlen, d):
    import numpy as np
    with jax.ensure_compile_time_eval():
        base = jax.random.key(0)
        kperm = jax.random.fold_in(base, 1)
        kmask = jax.random.fold_in(base, 2)
        permutation = jax.random.permutation(kperm, seqlen)
        area_mask = jax.random.bernoulli(
            kmask, 0.5, (bsz, d)).astype(jnp.float32)
        gidx = (jnp.arange(bsz, dtype=jnp.int32)[:, None] * seqlen
                + permutation.astype(jnp.int32)[None, :]
                ).reshape(_NW * _NHALF, _HR)
        return np.asarray(gidx), np.asarray(area_mask)


def kernel(x):
    bsz, seqlen, d = x.shape
    gidx, area_mask = _constants(bsz, seqlen, d)
    x2 = x.reshape(bsz * seqlen, d)
    out2 = _build()(x2, gidx, area_mask)
    return out2.reshape(bsz, seqlen, d)


# speedup vs baseline: 1.1494x; 1.1494x over previous
import functools

import jax
import jax.numpy as jnp
from jax import lax
from jax.experimental import pallas as pl
from jax.experimental.pallas import tpu as pltpu
from jax.experimental.pallas import tpu_sc as plsc

_NC, _NS, _L = 2, 16, 16
_NW = _NC * _NS
_BSZ, _SEQ, _D = 4, 2048, 4096
_ROWS = _BSZ * _SEQ
_CH = 8
_NF = _D // _L
_WPB = _NW // _BSZ


@functools.cache
def _chain_chunks():
    import contextlib
    import numpy as np
    try:
        ctx = jax.default_device(jax.local_devices(backend="cpu")[0])
    except Exception:
        ctx = contextlib.nullcontext()
    with ctx, jax.ensure_compile_time_eval():
        base = jax.random.key(0)
        kperm = jax.random.fold_in(base, 1)
        kmask = jax.random.fold_in(base, 2)
        perm = np.asarray(jax.random.permutation(kperm, _SEQ))
        area_mask = np.asarray(jax.random.bernoulli(
            kmask, 0.5, (_BSZ, _D)).astype(jnp.float32))

    visited = np.zeros(_SEQ, dtype=bool)
    chunks = []
    for start in range(_SEQ):
        if visited[start]:
            continue
        cyc = []
        s = start
        while not visited[s]:
            visited[s] = True
            cyc.append(s)
            s = int(perm[s])
        lcyc = len(cyc)
        walk = lcyc + (-lcyc % _CH)
        for c0 in range(0, walk, _CH):
            chunks.append([cyc[(c0 + t) % lcyc] for t in range(_CH + 1)])
    ncb = len(chunks)
    ncb_pad = ncb + (-ncb % (2 * _WPB))
    chunks = chunks + chunks[:ncb_pad - ncb]
    nchw = ncb_pad // _WPB
    stride = nchw + (-nchw % 8)
    gidx = np.zeros((_NW * stride, _CH + 1), dtype=np.int32)
    for w in range(_NW):
        b = w // _WPB
        off = b * _SEQ
        wslice = chunks[(w % _WPB) * nchw:(w % _WPB + 1) * nchw]
        for i in range(stride):
            ch = wslice[min(i, nchw - 1)]
            gidx[w * stride + i] = [off + v for v in ch]
    return (gidx[:, :_CH].copy(), gidx[:, _CH:].copy(), area_mask,
            ncb_pad, stride)

_NCB = 272
_NCHW = _NCB // _WPB
_IDXSTRIDE = _NCHW + (-_NCHW % 8)


def _sc_body(x_hbm, gidx_hbm, ovidx_hbm, mask_hbm, out_hbm,
             idx_v, ovidx_v, mask_v, cbuf_a, cbuf_b, ovb,
             sem_ga, sem_gb, sem_v, sem_oa, sem_ob):
    cbufs = (cbuf_a, cbuf_b)
    sem_g = (sem_ga, sem_gb)
    sem_o = (sem_oa, sem_ob)

    wid = lax.axis_index("s") * _NC + lax.axis_index("c")
    batch = wid // _WPB
    wc0 = wid * _IDXSTRIDE

    pltpu.sync_copy(gidx_hbm.at[pl.ds(wc0, _IDXSTRIDE)], idx_v)
    pltpu.sync_copy(ovidx_hbm.at[pl.ds(wc0, _IDXSTRIDE)], ovidx_v)
    pltpu.sync_copy(mask_hbm.at[batch], mask_v)

    def patch(cb, ovb):
        @plsc.parallel_loop(0, _NF, unroll=2)
        def feat(f):
            m = mask_v[pl.ds(f * _L, _L)]
            pred = m < 0.5
            col = lax.iota(jnp.int32, _L) + f * _L
            gs = [cb[k + 1, pl.ds(f * _L, _L)] for k in range(_CH - 1)]
            gs.append(ovb[0, pl.ds(f * _L, _L)])
            for k in range(_CH):
                row = jnp.full((_L,), k, dtype=jnp.int32)
                plsc.store_scatter(cb, [row, col], gs[k], mask=pred)

    def gather_idx(c):
        return idx_v.at[c]

    def scat_idx(c):
        return idx_v.at[c]

    def issue_gather(c, bs):
        pltpu.async_copy(x_hbm.at[gather_idx(c)], cbufs[bs], sem_g[bs])
        pltpu.async_copy(x_hbm.at[ovidx_v.at[c]], ovb, sem_v)

    issue_gather(0, 0)

    def pair(i, carry):
        for bs in (0, 1):
            c = 2 * i + bs
            nb = 1 - bs
            pltpu.make_async_copy(
                x_hbm.at[gather_idx(c)], cbufs[bs], sem_g[bs]).wait()
            pltpu.make_async_copy(
                x_hbm.at[ovidx_v.at[c]], ovb, sem_v).wait()
            patch(cbufs[bs], ovb)
            pltpu.async_copy(cbufs[bs], out_hbm.at[scat_idx(c)], sem_o[bs])

            @pl.when(c + 1 < _NCHW)
            def _issue_next():
                @pl.when(c >= 1)
                def _drain_prev_out():
                    pltpu.make_async_copy(
                        cbufs[nb],
                        out_hbm.at[scat_idx(c - 1)], sem_o[nb]).wait()
                issue_gather(c + 1, nb)
        return carry

    lax.fori_loop(0, _NCHW // 2, pair, 0)

    pltpu.make_async_copy(cbufs[0],
                          out_hbm.at[scat_idx(_NCHW - 2)], sem_o[0]).wait()
    pltpu.make_async_copy(cbufs[1],
                          out_hbm.at[scat_idx(_NCHW - 1)], sem_o[1]).wait()


@functools.cache
def _build():
    mesh = plsc.VectorSubcoreMesh(core_axis_name="c", subcore_axis_name="s")
    return pl.kernel(
        _sc_body,
        out_type=jax.ShapeDtypeStruct((_ROWS, _D), jnp.float32),
        mesh=mesh,
        scratch_types=[
            pltpu.VMEM((_IDXSTRIDE, _CH), jnp.int32),
            pltpu.VMEM((_IDXSTRIDE, 1), jnp.int32),
            pltpu.VMEM((_D,), jnp.float32),
            pltpu.VMEM((_CH, _D), jnp.float32),
            pltpu.VMEM((_CH, _D), jnp.float32),
            pltpu.VMEM((1, _D), jnp.float32),
            pltpu.SemaphoreType.DMA,
            pltpu.SemaphoreType.DMA,
            pltpu.SemaphoreType.DMA,
            pltpu.SemaphoreType.DMA,
            pltpu.SemaphoreType.DMA,
        ],
        compiler_params=pltpu.CompilerParams(
            use_tc_tiling_on_sc=True, needs_layout_passes=False),
    )


def kernel(x):
    bsz, seqlen, d = x.shape
    gidx, ovidx, mask, ncb, stride = _chain_chunks()
    assert ncb == _NCB and stride == _IDXSTRIDE
    x2 = x.reshape(bsz * seqlen, d)
    out2 = _build()(x2, gidx, ovidx, mask)
    return out2.reshape(bsz, seqlen, d)


# speedup vs baseline: 1.4340x; 1.2476x over previous
import functools

import jax
import jax.numpy as jnp
from jax import lax
from jax.experimental import pallas as pl
from jax.experimental.pallas import tpu as pltpu
from jax.experimental.pallas import tpu_sc as plsc

_NC, _NS, _L = 2, 16, 16
_NW = _NC * _NS
_BSZ, _SEQ, _D = 4, 2048, 4096
_ROWS = _BSZ * _SEQ
_CH = 8
_NF = _D // _L
_WPB = _NW // _BSZ


@functools.cache
def _chain_chunks():
    import contextlib
    import numpy as np
    try:
        ctx = jax.default_device(jax.local_devices(backend="cpu")[0])
    except Exception:
        ctx = contextlib.nullcontext()
    with ctx, jax.ensure_compile_time_eval():
        base = jax.random.key(0)
        kperm = jax.random.fold_in(base, 1)
        kmask = jax.random.fold_in(base, 2)
        perm = np.asarray(jax.random.permutation(kperm, _SEQ))
        area_mask = np.asarray(jax.random.bernoulli(
            kmask, 0.5, (_BSZ, _D)).astype(jnp.float32))

    visited = np.zeros(_SEQ, dtype=bool)
    chunks = []
    for start in range(_SEQ):
        if visited[start]:
            continue
        cyc = []
        s = start
        while not visited[s]:
            visited[s] = True
            cyc.append(s)
            s = int(perm[s])
        lcyc = len(cyc)
        walk = lcyc + (-lcyc % _CH)
        for c0 in range(0, walk, _CH):
            chunks.append([cyc[(c0 + t) % lcyc] for t in range(_CH + 1)])
    ncb = len(chunks)
    ncb_pad = ncb + (-ncb % (2 * _WPB))
    chunks = chunks + chunks[:ncb_pad - ncb]
    nchw = ncb_pad // _WPB
    stride = nchw + (-nchw % 8)
    gidx = np.zeros((_NW * stride, _CH + 1), dtype=np.int32)
    for w in range(_NW):
        b = w // _WPB
        off = b * _SEQ
        wslice = chunks[(w % _WPB) * nchw:(w % _WPB + 1) * nchw]
        for i in range(stride):
            ch = wslice[min(i, nchw - 1)]
            gidx[w * stride + i] = [off + v for v in ch]
    return (gidx[:, :_CH].copy(), gidx[:, _CH:].copy(), area_mask,
            ncb_pad, stride)

_NCB = 272
_NCHW = _NCB // _WPB
_IDXSTRIDE = _NCHW + (-_NCHW % 8)


def _sc_body(x_hbm, gidx_hbm, ovidx_hbm, mask_hbm, out_hbm,
             idx_v, ovidx_v, mask_v, cbuf_a, cbuf_b, ovb,
             sem_ga, sem_gb, sem_v, sem_oa, sem_ob):
    cbufs = (cbuf_a, cbuf_b)
    sem_g = (sem_ga, sem_gb)
    sem_o = (sem_oa, sem_ob)

    wid = lax.axis_index("s") * _NC + lax.axis_index("c")
    batch = wid // _WPB
    wc0 = wid * _IDXSTRIDE

    pltpu.sync_copy(gidx_hbm.at[pl.ds(wc0, _IDXSTRIDE)], idx_v)
    pltpu.sync_copy(ovidx_hbm.at[pl.ds(wc0, _IDXSTRIDE)], ovidx_v)
    pltpu.sync_copy(mask_hbm.at[batch], mask_v)

    def patch(cb, ovb):
        @plsc.parallel_loop(0, _NF, unroll=2)
        def feat(f):
            m = mask_v[pl.ds(f * _L, _L)]
            pred = m < 0.5
            col = lax.iota(jnp.int32, _L) + f * _L
            gs = [cb[k + 1, pl.ds(f * _L, _L)] for k in range(_CH - 1)]
            gs.append(ovb[0, pl.ds(f * _L, _L)])
            for k in range(_CH):
                row = jnp.full((_L,), k, dtype=jnp.int32)
                plsc.store_scatter(cb, [row, col], gs[k], mask=pred)

    def gather_idx(c):
        return idx_v.at[c]

    def scat_idx(c):
        return idx_v.at[c]

    def issue_gather(c, bs):
        pltpu.async_copy(x_hbm.at[gather_idx(c)], cbufs[bs], sem_g[bs])

    def issue_ov(c):
        pltpu.async_copy(x_hbm.at[ovidx_v.at[c]], ovb, sem_v)

    issue_gather(0, 0)
    issue_ov(0)

    def pair(i, carry):
        for bs in (0, 1):
            c = 2 * i + bs
            nb = 1 - bs
            pltpu.make_async_copy(
                x_hbm.at[gather_idx(c)], cbufs[bs], sem_g[bs]).wait()

            @pl.when(c + 1 < _NCHW)
            def _issue_next_main():
                @pl.when(c >= 1)
                def _drain_prev_out():
                    pltpu.make_async_copy(
                        cbufs[nb],
                        out_hbm.at[scat_idx(c - 1)], sem_o[nb]).wait()
                issue_gather(c + 1, nb)

            pltpu.make_async_copy(
                x_hbm.at[ovidx_v.at[c]], ovb, sem_v).wait()
            patch(cbufs[bs], ovb)
            pltpu.async_copy(cbufs[bs], out_hbm.at[scat_idx(c)], sem_o[bs])

            @pl.when(c + 1 < _NCHW)
            def _issue_next_ov():
                issue_ov(c + 1)
        return carry

    lax.fori_loop(0, _NCHW // 2, pair, 0)

    pltpu.make_async_copy(cbufs[0],
                          out_hbm.at[scat_idx(_NCHW - 2)], sem_o[0]).wait()
    pltpu.make_async_copy(cbufs[1],
                          out_hbm.at[scat_idx(_NCHW - 1)], sem_o[1]).wait()


@functools.cache
def _build():
    mesh = plsc.VectorSubcoreMesh(core_axis_name="c", subcore_axis_name="s")
    return pl.kernel(
        _sc_body,
        out_type=jax.ShapeDtypeStruct((_ROWS, _D), jnp.float32),
        mesh=mesh,
        scratch_types=[
            pltpu.VMEM((_IDXSTRIDE, _CH), jnp.int32),
            pltpu.VMEM((_IDXSTRIDE, 1), jnp.int32),
            pltpu.VMEM((_D,), jnp.float32),
            pltpu.VMEM((_CH, _D), jnp.float32),
            pltpu.VMEM((_CH, _D), jnp.float32),
            pltpu.VMEM((1, _D), jnp.float32),
            pltpu.SemaphoreType.DMA,
            pltpu.SemaphoreType.DMA,
            pltpu.SemaphoreType.DMA,
            pltpu.SemaphoreType.DMA,
            pltpu.SemaphoreType.DMA,
        ],
        compiler_params=pltpu.CompilerParams(
            use_tc_tiling_on_sc=True, needs_layout_passes=False),
    )


def kernel(x):
    bsz, seqlen, d = x.shape
    gidx, ovidx, mask, ncb, stride = _chain_chunks()
    assert ncb == _NCB and stride == _IDXSTRIDE
    x2 = x.reshape(bsz * seqlen, d)
    out2 = _build()(x2, gidx, ovidx, mask)
    return out2.reshape(bsz, seqlen, d)
